# baseline (speedup 1.0000x reference)
import functools

import jax
import jax.numpy as jnp
from jax import lax
from jax.experimental import pallas as pl
from jax.experimental.pallas import tpu as pltpu
from jax.experimental.pallas import tpu_sc as plsc

LANES = 16


def _tw_tc_kernel(table_ref, w_ref, b_ref, out_ref, *, inv_l):
    t = table_ref[...]
    w = w_ref[...]
    tw = jnp.dot(t, w, preferred_element_type=jnp.float32)
    out_ref[...] = jnp.broadcast_to((tw + b_ref[0]) * inv_l,
                                    out_ref.shape)


def _make_sc_lookup(V_pad, B, L, n_workers, chunk_sents):
    sents_per_worker = B // n_workers
    n_chunks = sents_per_worker // chunk_sents
    blocks_per_chunk = chunk_sents // LANES
    mesh = plsc.VectorSubcoreMesh(core_axis_name="c", subcore_axis_name="s")

    @functools.partial(
        pl.kernel,
        mesh=mesh,
        out_type=jax.ShapeDtypeStruct((B,), jnp.float32),
        scratch_types=[
            pltpu.VMEM((chunk_sents * L,), jnp.int32),
            pltpu.VMEM((V_pad * LANES,), jnp.float32),
            pltpu.VMEM((sents_per_worker,), jnp.float32),
        ],
        compiler_params=pltpu.CompilerParams(needs_layout_passes=False),
    )
    def sc_lookup(x_hbm, twi_hbm, out_hbm, idx_v, twi_v, out_v):
        n_cores = 2
        wid = lax.axis_index("s") * n_cores + lax.axis_index("c")
        base_s = wid * sents_per_worker

        pltpu.sync_copy(twi_hbm, twi_v)
        j16 = lax.iota(jnp.int32, LANES)

        for c in range(n_chunks):
            start = (base_s + c * chunk_sents) * L
            pltpu.sync_copy(x_hbm.at[pl.ds(start, chunk_sents * L)], idx_v)
            for blk in range(blocks_per_chunk):
                rowbase = (blk * LANES + j16) * L
                rowend = rowbase + L
                addr0 = rowbase + j16

                def body(t, carry, rowend=rowend):
                    addr, acc = carry
                    xv = plsc.load_gather(idx_v, [addr])
                    tv = plsc.load_gather(twi_v, [(xv << 4) + j16])
                    addr = addr + 1
                    addr = jnp.where(addr >= rowend, addr - L, addr)
                    return addr, acc + tv

                _, acc = lax.fori_loop(
                    0, L, body,
                    (addr0, jnp.zeros((LANES,), jnp.float32)),
                    unroll=8)
                pred = 1.0 / (1.0 + jnp.exp(-acc))
                out_v[pl.ds(c * chunk_sents + blk * LANES, LANES)] = pred

        pltpu.sync_copy(out_v, out_hbm.at[pl.ds(base_s, sents_per_worker)])

    return sc_lookup


def kernel(x, table, W, b):
    B, L = x.shape
    V, EMB = table.shape
    V_pad = ((V + 7) // 8) * 8

    table_p = jnp.pad(table, ((0, V_pad - V), (0, 0)))
    twi = pl.pallas_call(
        functools.partial(_tw_tc_kernel, inv_l=1.0 / L),
        out_shape=jax.ShapeDtypeStruct((V_pad, LANES), jnp.float32),
    )(table_p, W, b)

    x_flat = x.reshape(-1).astype(jnp.int32)
    out = _make_sc_lookup(V_pad, B, L, 32, 512)(x_flat, twi.reshape(-1))
    return out.reshape(B, 1)


# speedup vs baseline: 1.2329x; 1.2329x over previous
import functools

import jax
import jax.numpy as jnp
from jax import lax
from jax.experimental import pallas as pl
from jax.experimental.pallas import tpu as pltpu
from jax.experimental.pallas import tpu_sc as plsc

LANES = 16


def _tw_tc_kernel(table_ref, w_ref, b_ref, out_ref, *, inv_l):
    t = table_ref[...]
    w = w_ref[...]
    tw = jnp.dot(t, w, preferred_element_type=jnp.float32)
    out_ref[...] = (tw + b_ref[0]) * inv_l


def _make_sc_lookup(V_pad, B, L, n_workers, chunk_sents):
    sents_per_worker = B // n_workers
    n_chunks = sents_per_worker // chunk_sents
    blocks_per_chunk = chunk_sents // LANES
    mesh = plsc.VectorSubcoreMesh(core_axis_name="c", subcore_axis_name="s")

    @functools.partial(
        pl.kernel,
        mesh=mesh,
        out_type=jax.ShapeDtypeStruct((B,), jnp.float32),
        scratch_types=[
            pltpu.VMEM((chunk_sents, L), jnp.int32),
            pltpu.VMEM((V_pad,), jnp.float32),
            pltpu.VMEM((sents_per_worker,), jnp.float32),
        ],
        compiler_params=pltpu.CompilerParams(needs_layout_passes=False),
    )
    def sc_lookup(x_hbm, tw_hbm, out_hbm, idx_v, tw_v, out_v):
        n_cores = 2
        wid = lax.axis_index("s") * n_cores + lax.axis_index("c")
        base_s = wid * sents_per_worker

        pltpu.sync_copy(tw_hbm, tw_v)
        j16 = lax.iota(jnp.int32, LANES)

        for c in range(n_chunks):
            pltpu.sync_copy(
                x_hbm.at[pl.ds(base_s + c * chunk_sents, chunk_sents), :],
                idx_v)
            for blk in range(blocks_per_chunk):
                rvec = blk * LANES + j16

                def body(t, carry, rvec=rvec):
                    cvec, acc = carry
                    xv = plsc.load_gather(idx_v, [rvec, cvec])
                    tv = plsc.load_gather(tw_v, [xv])
                    cvec = cvec + 1
                    cvec = jnp.where(cvec >= L, cvec - L, cvec)
                    return cvec, acc + tv

                _, acc = lax.fori_loop(
                    0, L, body,
                    (j16, jnp.zeros((LANES,), jnp.float32)),
                    unroll=8)
                pred = 1.0 / (1.0 + jnp.exp(-acc))
                out_v[pl.ds(c * chunk_sents + blk * LANES, LANES)] = pred

        pltpu.sync_copy(out_v, out_hbm.at[pl.ds(base_s, sents_per_worker)])

    return sc_lookup


def kernel(x, table, W, b):
    B, L = x.shape
    V, EMB = table.shape
    V_pad = ((V + 7) // 8) * 8

    table_p = jnp.pad(table, ((0, V_pad - V), (0, 0)))
    tw = pl.pallas_call(
        functools.partial(_tw_tc_kernel, inv_l=1.0 / L),
        out_shape=jax.ShapeDtypeStruct((V_pad, 1), jnp.float32),
    )(table_p, W, b)

    out = _make_sc_lookup(V_pad, B, L, 32, 256)(
        x.astype(jnp.int32), tw.reshape(-1))
    return out.reshape(B, 1)


# speedup vs baseline: 2.0901x; 1.6954x over previous
import functools

import jax
import jax.numpy as jnp
from jax import lax
from jax.experimental import pallas as pl
from jax.experimental.pallas import tpu as pltpu
from jax.experimental.pallas import tpu_sc as plsc

LANES = 16


def _tw_tc_kernel(tableT_ref, wT_ref, b_ref, out_ref, *, inv_l):
    tT = tableT_ref[...]
    wT = wT_ref[...]
    tw = jnp.dot(wT, tT, preferred_element_type=jnp.float32)
    out_ref[...] = (tw + b_ref[0]) * inv_l


def _make_sc_lookup(V, B, L, n_workers):
    sents_per_worker = B // n_workers
    n_blocks = sents_per_worker // LANES
    mesh = plsc.VectorSubcoreMesh(core_axis_name="c", subcore_axis_name="s")

    @functools.partial(
        pl.kernel,
        mesh=mesh,
        out_type=jax.ShapeDtypeStruct((B,), jnp.float32),
        scratch_types=[
            pltpu.VMEM((L, sents_per_worker), jnp.int32),
            pltpu.VMEM((V,), jnp.float32),
            pltpu.VMEM((sents_per_worker,), jnp.float32),
        ],
        compiler_params=pltpu.CompilerParams(needs_layout_passes=False),
    )
    def sc_lookup(xt_hbm, tw_hbm, out_hbm, idx_v, tw_v, out_v):
        n_cores = 2
        wid = lax.axis_index("s") * n_cores + lax.axis_index("c")
        base_s = wid * sents_per_worker

        pltpu.sync_copy(tw_hbm.at[0], tw_v)
        pltpu.sync_copy(xt_hbm.at[:, pl.ds(base_s, sents_per_worker)], idx_v)

        for blk in range(n_blocks):
            s0 = blk * LANES

            def body(t, acc, s0=s0):
                xv = idx_v[t, pl.ds(s0, LANES)]
                tv = plsc.load_gather(tw_v, [xv])
                return acc + tv

            acc = lax.fori_loop(0, L, body,
                                jnp.zeros((LANES,), jnp.float32),
                                unroll=8)
            pred = 1.0 / (1.0 + jnp.exp(-acc))
            out_v[pl.ds(s0, LANES)] = pred

        pltpu.sync_copy(out_v, out_hbm.at[pl.ds(base_s, sents_per_worker)])

    return sc_lookup


def kernel(x, table, W, b):
    B, L = x.shape
    V, EMB = table.shape

    tw = pl.pallas_call(
        functools.partial(_tw_tc_kernel, inv_l=1.0 / L),
        out_shape=jax.ShapeDtypeStruct((1, V), jnp.float32),
    )(table.T, W.T, b)

    out = _make_sc_lookup(V, B, L, 32)(x.T.astype(jnp.int32), tw)
    return out.reshape(B, 1)


# speedup vs baseline: 2.2685x; 1.0854x over previous
import functools

import jax
import jax.numpy as jnp
from jax import lax
from jax.experimental import pallas as pl
from jax.experimental.pallas import tpu as pltpu
from jax.experimental.pallas import tpu_sc as plsc

LANES = 16


def _tw_tc_kernel(tableT_ref, wT_ref, b_ref, out_ref, *, inv_l):
    tT = tableT_ref[...]
    wT = wT_ref[...]
    tw = jnp.dot(wT, tT, preferred_element_type=jnp.float32)
    out_ref[...] = (tw + b_ref[0]) * inv_l


def _make_sc_lookup(V, B, L, n_workers):
    sents_per_worker = B // n_workers
    n_blocks = sents_per_worker // LANES
    mesh = plsc.VectorSubcoreMesh(core_axis_name="c", subcore_axis_name="s")

    @functools.partial(
        pl.kernel,
        mesh=mesh,
        out_type=jax.ShapeDtypeStruct((B,), jnp.float32),
        scratch_types=[
            pltpu.VMEM((L, sents_per_worker), jnp.int32),
            pltpu.VMEM((V,), jnp.float32),
            pltpu.VMEM((sents_per_worker,), jnp.float32),
        ],
        compiler_params=pltpu.CompilerParams(needs_layout_passes=False),
    )
    def sc_lookup(xt_hbm, tw_hbm, out_hbm, idx_v, tw_v, out_v):
        n_cores = 2
        wid = lax.axis_index("s") * n_cores + lax.axis_index("c")
        base_s = wid * sents_per_worker

        pltpu.sync_copy(tw_hbm.at[0], tw_v)
        pltpu.sync_copy(xt_hbm.at[:, pl.ds(base_s, sents_per_worker)], idx_v)

        def blk_body(blk, _):
            s0 = blk * LANES

            def body(t, acc):
                xv = idx_v[t, pl.ds(s0, LANES)]
                tv = plsc.load_gather(tw_v, [xv])
                return acc + tv

            acc = lax.fori_loop(0, L, body,
                                jnp.zeros((LANES,), jnp.float32),
                                unroll=8)
            pred = 1.0 / (1.0 + jnp.exp(-acc))
            out_v[pl.ds(s0, LANES)] = pred
            return 0

        lax.fori_loop(0, n_blocks, blk_body, 0)

        pltpu.sync_copy(out_v, out_hbm.at[pl.ds(base_s, sents_per_worker)])

    return sc_lookup


def kernel(x, table, W, b):
    B, L = x.shape
    V, EMB = table.shape

    tw = pl.pallas_call(
        functools.partial(_tw_tc_kernel, inv_l=1.0 / L),
        out_shape=jax.ShapeDtypeStruct((1, V), jnp.float32),
    )(table.T, W.T, b)

    out = _make_sc_lookup(V, B, L, 32)(x.T.astype(jnp.int32), tw)
    return out.reshape(B, 1)


# speedup vs baseline: 2.3134x; 1.0198x over previous
import functools

import jax
import jax.numpy as jnp
from jax import lax
from jax.experimental import pallas as pl
from jax.experimental.pallas import tpu as pltpu
from jax.experimental.pallas import tpu_sc as plsc

LANES = 16
N_XCHUNKS = 4


def _tw_tc_kernel(tableT_ref, wT_ref, b_ref, out_ref, *, inv_l):
    tT = tableT_ref[...]
    wT = wT_ref[...]
    tw = jnp.dot(wT, tT, preferred_element_type=jnp.float32)
    out_ref[...] = (tw + b_ref[0]) * inv_l


def _make_sc_lookup(V, B, L, n_workers):
    sents_per_worker = B // n_workers
    n_blocks = sents_per_worker // LANES
    mesh = plsc.VectorSubcoreMesh(core_axis_name="c", subcore_axis_name="s")

    @functools.partial(
        pl.kernel,
        mesh=mesh,
        out_type=jax.ShapeDtypeStruct((B,), jnp.float32),
        scratch_types=[
            pltpu.VMEM((L, sents_per_worker), jnp.int32),
            pltpu.VMEM((V,), jnp.float32),
            pltpu.VMEM((sents_per_worker,), jnp.float32),
            [pltpu.SemaphoreType.DMA] * N_XCHUNKS,
        ],
        compiler_params=pltpu.CompilerParams(needs_layout_passes=False),
    )
    def sc_lookup(xt_hbm, tw_hbm, out_hbm, idx_v, tw_v, out_v, sems):
        n_cores = 2
        wid = lax.axis_index("s") * n_cores + lax.axis_index("c")
        base_s = wid * sents_per_worker
        chunk = sents_per_worker // N_XCHUNKS

        copies = [
            pltpu.async_copy(
                xt_hbm.at[:, pl.ds(base_s + c * chunk, chunk)],
                idx_v.at[:, pl.ds(c * chunk, chunk)],
                sems[c])
            for c in range(N_XCHUNKS)
        ]
        pltpu.sync_copy(tw_hbm.at[0], tw_v)

        for c in range(N_XCHUNKS):
            copies[c].wait()

            def blk_body(blk, _):
                s0 = blk * LANES

                def body(t, acc):
                    xv = idx_v[t, pl.ds(s0, LANES)]
                    tv = plsc.load_gather(tw_v, [xv])
                    return acc + tv

                acc = lax.fori_loop(0, L, body,
                                    jnp.zeros((LANES,), jnp.float32),
                                    unroll=8)
                pred = 1.0 / (1.0 + jnp.exp(-acc))
                out_v[pl.ds(s0, LANES)] = pred
                return 0

            lax.fori_loop(c * (n_blocks // N_XCHUNKS),
                          (c + 1) * (n_blocks // N_XCHUNKS), blk_body, 0)

        pltpu.sync_copy(out_v, out_hbm.at[pl.ds(base_s, sents_per_worker)])

    return sc_lookup


def kernel(x, table, W, b):
    B, L = x.shape
    V, EMB = table.shape

    tw = pl.pallas_call(
        functools.partial(_tw_tc_kernel, inv_l=1.0 / L),
        out_shape=jax.ShapeDtypeStruct((1, V), jnp.float32),
    )(table.T, W.T, b)

    out = _make_sc_lookup(V, B, L, 32)(x.T.astype(jnp.int32), tw)
    return out.reshape(B, 1)
